# baseline (speedup 1.0000x reference)
import jax
import jax.numpy as jnp
from jax.experimental import pallas as pl
from jax.experimental.pallas import tpu as pltpu

N = 8192
TD = 768
ID = 768
H = 512
OD = 768
E = 8
NOISE_STD = 1.0

T = 512


def _fused_body(xt_ref, xi_ref, wt_ref, bt_ref, wi_ref, bi_ref,
                wg_ref, bg_ref, noise_ref, w1c_ref, b1c_ref, w2s_ref, b2_ref,
                out_ref):
    tp = jnp.dot(xt_ref[...], wt_ref[...], preferred_element_type=jnp.float32)
    tp = tp + bt_ref[...]
    ip = jnp.dot(xi_ref[...], wi_ref[...], preferred_element_type=jnp.float32)
    ip = ip + bi_ref[...]
    comb = jnp.concatenate([tp, ip], axis=1)

    logits = jnp.dot(comb, wg_ref[...], preferred_element_type=jnp.float32)
    logits = logits + bg_ref[...] + noise_ref[...] * NOISE_STD

    lane = jax.lax.broadcasted_iota(jnp.int32, (T, E), 1)
    m1 = jnp.max(logits, axis=1, keepdims=True)
    is1 = logits == m1
    idx1 = jnp.min(jnp.where(is1, lane, E), axis=1, keepdims=True)
    masked = jnp.where(lane == idx1, -jnp.inf, logits)
    m2 = jnp.max(masked, axis=1, keepdims=True)
    is2 = masked == m2
    idx2 = jnp.min(jnp.where(is2, lane, E), axis=1, keepdims=True)
    z = jnp.exp(m2 - m1)
    w1 = 1.0 / (1.0 + z)
    w2 = 1.0 - w1
    gates = jnp.where(lane == idx1, w1, jnp.where(lane == idx2, w2, 0.0))

    h = jnp.dot(comb.astype(jnp.bfloat16), w1c_ref[...],
                preferred_element_type=jnp.float32)
    h = jnp.maximum(h + b1c_ref[...], 0.0)
    col_e = jax.lax.broadcasted_iota(jnp.int32, (E, E * H), 1) // H
    row_e = jax.lax.broadcasted_iota(jnp.int32, (E, E * H), 0)
    sel = (col_e == row_e).astype(jnp.float32)
    ge = jnp.dot(gates, sel, preferred_element_type=jnp.float32)
    hg = (h * ge).astype(jnp.bfloat16)
    y = jnp.dot(hg, w2s_ref[...], preferred_element_type=jnp.float32)
    out_ref[...] = y + jnp.dot(gates, b2_ref[...],
                               preferred_element_type=jnp.float32)


def kernel(text_emb, image_emb, Wt, bt, Wi, bi, Wg, bg, W1, b1, W2, b2, noise):
    W1c = W1.transpose(1, 0, 2).reshape(2 * H, E * H).astype(jnp.bfloat16)
    b1c = b1.reshape(1, E * H)
    W2s = W2.reshape(E * H, OD).astype(jnp.bfloat16)

    out = pl.pallas_call(
        _fused_body,
        grid=(N // T,),
        in_specs=[
            pl.BlockSpec((T, TD), lambda t: (t, 0)),
            pl.BlockSpec((T, ID), lambda t: (t, 0)),
            pl.BlockSpec((TD, H), lambda t: (0, 0)),
            pl.BlockSpec((H,), lambda t: (0,)),
            pl.BlockSpec((ID, H), lambda t: (0, 0)),
            pl.BlockSpec((H,), lambda t: (0,)),
            pl.BlockSpec((2 * H, E), lambda t: (0, 0)),
            pl.BlockSpec((E,), lambda t: (0,)),
            pl.BlockSpec((T, E), lambda t: (t, 0)),
            pl.BlockSpec((2 * H, E * H), lambda t: (0, 0)),
            pl.BlockSpec((1, E * H), lambda t: (0, 0)),
            pl.BlockSpec((E * H, OD), lambda t: (0, 0)),
            pl.BlockSpec((E, OD), lambda t: (0, 0)),
        ],
        out_specs=pl.BlockSpec((T, OD), lambda t: (t, 0)),
        out_shape=jax.ShapeDtypeStruct((N, OD), jnp.float32),
        compiler_params=pltpu.CompilerParams(
            dimension_semantics=("arbitrary",)),
    )(text_emb, image_emb, Wt, bt, Wi, bi, Wg, bg, noise, W1c, b1c, W2s, b2)
    return out


# speedup vs baseline: 1.3147x; 1.3147x over previous
import jax
import jax.numpy as jnp
from jax.experimental import pallas as pl
from jax.experimental.pallas import tpu as pltpu

N = 8192
TD = 768
ID = 768
H = 512
OD = 768
E = 8
NOISE_STD = 1.0

TA = 512
TB = 512


def _proj_gate_body(xt_ref, xi_ref, wt_ref, bt_ref, wi_ref, bi_ref,
                    wg_ref, bg_ref, noise_ref, comb_ref, gates_ref):
    tp = jnp.dot(xt_ref[...], wt_ref[...], preferred_element_type=jnp.float32)
    tp = tp + bt_ref[...]
    ip = jnp.dot(xi_ref[...], wi_ref[...], preferred_element_type=jnp.float32)
    ip = ip + bi_ref[...]
    comb = jnp.concatenate([tp, ip], axis=1)
    comb_ref[...] = comb.astype(jnp.bfloat16)

    logits = jnp.dot(comb, wg_ref[...], preferred_element_type=jnp.float32)
    logits = logits + bg_ref[...] + noise_ref[...] * NOISE_STD

    lane = jax.lax.broadcasted_iota(jnp.int32, (TA, E), 1)
    m1 = jnp.max(logits, axis=1, keepdims=True)
    is1 = logits == m1
    idx1 = jnp.min(jnp.where(is1, lane, E), axis=1, keepdims=True)
    masked = jnp.where(lane == idx1, -jnp.inf, logits)
    m2 = jnp.max(masked, axis=1, keepdims=True)
    is2 = masked == m2
    idx2 = jnp.min(jnp.where(is2, lane, E), axis=1, keepdims=True)
    z = jnp.exp(m2 - m1)
    w1 = 1.0 / (1.0 + z)
    w2 = 1.0 - w1
    gates_ref[...] = jnp.where(lane == idx1, w1,
                               jnp.where(lane == idx2, w2, 0.0))


def _moe_body(comb_ref, gates_ref, w1_ref, b1_ref, w2_ref, b2_ref, out_ref,
              w1bf_ref, w2bf_ref, hg_ref):
    t = pl.program_id(0)

    @pl.when(t == 0)
    def _():
        w1bf_ref[...] = w1_ref[...].astype(jnp.bfloat16)
        w2bf_ref[...] = w2_ref[...].reshape(E * H, OD).astype(jnp.bfloat16)

    x = comb_ref[...]
    gates = gates_ref[...]
    lane = jax.lax.broadcasted_iota(jnp.int32, (TB, E), 1)
    for e in range(E):
        he = jnp.dot(x, w1bf_ref[e], preferred_element_type=jnp.float32)
        ge = jnp.sum(jnp.where(lane == e, gates, 0.0), axis=1, keepdims=True)
        hg_ref[:, e * H:(e + 1) * H] = (
            jnp.maximum(he + b1_ref[e], 0.0) * ge).astype(jnp.bfloat16)
    y = jnp.dot(hg_ref[...], w2bf_ref[...], preferred_element_type=jnp.float32)
    out_ref[...] = y + jnp.dot(gates, b2_ref[...],
                               preferred_element_type=jnp.float32)


def kernel(text_emb, image_emb, Wt, bt, Wi, bi, Wg, bg, W1, b1, W2, b2, noise):
    comb, gates = pl.pallas_call(
        _proj_gate_body,
        grid=(N // TA,),
        in_specs=[
            pl.BlockSpec((TA, TD), lambda t: (t, 0)),
            pl.BlockSpec((TA, ID), lambda t: (t, 0)),
            pl.BlockSpec((TD, H), lambda t: (0, 0)),
            pl.BlockSpec((H,), lambda t: (0,)),
            pl.BlockSpec((ID, H), lambda t: (0, 0)),
            pl.BlockSpec((H,), lambda t: (0,)),
            pl.BlockSpec((2 * H, E), lambda t: (0, 0)),
            pl.BlockSpec((E,), lambda t: (0,)),
            pl.BlockSpec((TA, E), lambda t: (t, 0)),
        ],
        out_specs=[
            pl.BlockSpec((TA, 2 * H), lambda t: (t, 0)),
            pl.BlockSpec((TA, E), lambda t: (t, 0)),
        ],
        out_shape=[
            jax.ShapeDtypeStruct((N, 2 * H), jnp.bfloat16),
            jax.ShapeDtypeStruct((N, E), jnp.float32),
        ],
        compiler_params=pltpu.CompilerParams(
            dimension_semantics=("arbitrary",)),
    )(text_emb, image_emb, Wt, bt, Wi, bi, Wg, bg, noise)

    out = pl.pallas_call(
        _moe_body,
        grid=(N // TB,),
        in_specs=[
            pl.BlockSpec((TB, 2 * H), lambda t: (t, 0)),
            pl.BlockSpec((TB, E), lambda t: (t, 0)),
            pl.BlockSpec((E, 2 * H, H), lambda t: (0, 0, 0)),
            pl.BlockSpec((E, H), lambda t: (0, 0)),
            pl.BlockSpec((E, H, OD), lambda t: (0, 0, 0)),
            pl.BlockSpec((E, OD), lambda t: (0, 0)),
        ],
        out_specs=pl.BlockSpec((TB, OD), lambda t: (t, 0)),
        out_shape=jax.ShapeDtypeStruct((N, OD), jnp.float32),
        scratch_shapes=[
            pltpu.VMEM((E, 2 * H, H), jnp.bfloat16),
            pltpu.VMEM((E * H, OD), jnp.bfloat16),
            pltpu.VMEM((TB, E * H), jnp.bfloat16),
        ],
        compiler_params=pltpu.CompilerParams(
            dimension_semantics=("arbitrary",)),
    )(comb, gates, W1, b1, W2, b2)
    return out
